# baseline (speedup 1.0000x reference)
import jax
import jax.numpy as jnp
from jax.experimental import pallas as pl
from jax.experimental.pallas import tpu as pltpu

EPS = 1e-6
TJ = 256


def _state_kernel(wkeyt_ref, wvec_ref, erase_ref, free_ref, rstr_ref,
                  scal_ref, rkeys_ref, memf_ref, rw0_ref, rw1_ref, rw2_ref,
                  rw3_ref, ww_ref, usage_ref, pw_ref,
                  memnewf_ref, w2_ref, usageout_ref, precout_ref,
                  raddrt_ref):
    B, M = ww_ref.shape
    A = memf_ref.shape[1]
    R = rkeys_ref.shape[2]

    ws_col = jax.nn.softplus(scal_ref[:, 0:1]) + 1.0
    ag_col = jax.nn.sigmoid(scal_ref[:, 1:2])
    wg_col = jax.nn.sigmoid(scal_ref[:, 2:3])
    fg = jax.nn.sigmoid(free_ref[:])

    rws = (rw0_ref[:], rw1_ref[:], rw2_ref[:], rw3_ref[:])
    retention = 1.0 - fg[:, 0:1] * rws[0]
    for r in range(1, R):
        retention = retention * (1.0 - fg[:, r:r + 1] * rws[r])
    prev_u = usage_ref[:]
    ww = ww_ref[:]
    usage = ((prev_u + ww) - prev_u * ww) * retention
    usageout_ref[:] = usage

    u = EPS + (1.0 - EPS) * usage
    logu = jnp.log(u)
    ii = jax.lax.broadcasted_iota(jnp.int32, (M, M), 0)
    jj = jax.lax.broadcasted_iota(jnp.int32, (M, M), 1)
    ltm = jj < ii
    alloc_rows = []
    for b in range(B):
        u_row = u[b:b + 1, :]
        u_col = jnp.transpose(u_row)
        logu_col = jnp.transpose(logu[b:b + 1, :])
        mask = (u_row < u_col) | ((u_row == u_col) & ltm)
        s = jnp.dot(mask.astype(jnp.float32), logu_col,
                    preferred_element_type=jnp.float32)
        alloc_rows.append(jnp.transpose(s))
    s_all = jnp.concatenate(alloc_rows, axis=0)
    alloc = (1.0 - u) * jnp.exp(s_all)

    memf = memf_ref[:]
    dots_all = jnp.dot(memf, wkeyt_ref[:],
                       preferred_element_type=jnp.float32)
    key_norm = jnp.sqrt(jnp.sum(wkeyt_ref[:] * wkeyt_ref[:], axis=0,
                                keepdims=True))
    msq = jnp.sum(memf * memf, axis=1, keepdims=True)
    dot_rows, norm_rows = [], []
    for b in range(B):
        dot_rows.append(jnp.transpose(dots_all[b * M:(b + 1) * M, b:b + 1]))
        norm_rows.append(jnp.transpose(msq[b * M:(b + 1) * M, :]))
    dots = jnp.concatenate(dot_rows, axis=0)
    mem_norm = jnp.sqrt(jnp.concatenate(norm_rows, axis=0))
    sim = dots / (mem_norm * jnp.transpose(key_norm) + EPS) * ws_col
    write_addr = jax.nn.softmax(sim, axis=1)

    write_w = wg_col * ((1.0 - ag_col) * write_addr + ag_col * alloc)
    w2_ref[:] = write_w

    precout_ref[:] = ((1.0 - jnp.sum(write_w, axis=1, keepdims=True))
                      * pw_ref[:] + write_w)

    erase = jax.nn.sigmoid(erase_ref[:])
    rstr = jax.nn.softplus(rstr_ref[:]) + 1.0
    for b in range(B):
        w_col = jnp.transpose(write_w[b:b + 1, :])
        mem_b = memf[b * M:(b + 1) * M, :]
        mem_new = (mem_b * (1.0 - w_col * erase[b:b + 1, :])
                   + w_col * wvec_ref[b:b + 1, :])
        memnewf_ref[b * M:(b + 1) * M, :] = mem_new

        rkeys = rkeys_ref[b]
        dotr = jnp.dot(mem_new, rkeys,
                       preferred_element_type=jnp.float32)
        nsq = jnp.sum(mem_new * mem_new, axis=1, keepdims=True)
        dotr_t = jnp.transpose(dotr)
        norm_t = jnp.sqrt(jnp.transpose(nsq))
        rkn_col = jnp.sqrt(jnp.sum(rkeys * rkeys, axis=0,
                                   keepdims=True)).reshape(R, 1)
        simr_t = (dotr_t / (norm_t * rkn_col + EPS)
                  * jnp.transpose(rstr[b:b + 1, :]))
        raddrt_ref[b] = jax.nn.softmax(simr_t, axis=1)


def _link_kernel(link_ref, wcol_ref, wrow_ref, pwrow_ref, rw_ref,
                 memnew_ref, raddrt_ref, rmraw_ref, wperm_ref, bout_ref,
                 lnew_ref, readw_ref, readvec_ref, out_ref,
                 fwd_s, bwd_s):
    j = pl.program_id(1)
    nj = pl.num_programs(1)
    M = rw_ref.shape[1]
    R = rw_ref.shape[2]
    tj = link_ref.shape[1]

    L = link_ref[0]
    w_row = wrow_ref[0]
    pw_row = pwrow_ref[0]
    wJ = wcol_ref[0, pl.ds(j * tj, tj), :]
    rw_full = rw_ref[0]

    lnew = (1.0 - wJ + w_row) * L + wJ * pw_row
    row_g = jax.lax.broadcasted_iota(jnp.int32, (tj, M), 0) + j * tj
    col_g = jax.lax.broadcasted_iota(jnp.int32, (tj, M), 1)
    lnew = jnp.where(row_g == col_g, 0.0, lnew)
    lnew_ref[0] = lnew

    fwd_s[pl.ds(j * tj, tj), :] = jnp.dot(
        lnew, rw_full, preferred_element_type=jnp.float32)

    rwJ = rw_ref[0, pl.ds(j * tj, tj), :]
    contrib = jax.lax.dot_general(rwJ, lnew, (((0,), (0,)), ((), ())),
                                  preferred_element_type=jnp.float32)

    @pl.when(j == 0)
    def _():
        bwd_s[:] = contrib

    @pl.when(j != 0)
    def _():
        bwd_s[:] += contrib

    @pl.when(j == nj - 1)
    def _stage_c():
        rm = jax.nn.softmax(rmraw_ref[0], axis=0)
        rm_col = jnp.transpose(rm)
        read_w_t = (bwd_s[:] * rm_col[:, 0:1]
                    + raddrt_ref[0] * rm_col[:, 1:2]
                    + jnp.transpose(fwd_s[:]) * rm_col[:, 2:3])
        readw_ref[0] = jnp.transpose(read_w_t)
        rv_t = jax.lax.dot_general(read_w_t, memnew_ref[0],
                                   (((1,), (0,)), ((), ())),
                                   preferred_element_type=jnp.float32)
        readvec_ref[0] = jnp.transpose(rv_t)
        acc = bout_ref[:]
        for r in range(R):
            acc = acc + jax.lax.dot_general(
                rv_t[r:r + 1, :], wperm_ref[r],
                (((1,), (0,)), ((), ())),
                preferred_element_type=jnp.float32)
        out_ref[0] = acc


def kernel(interface, memory, read_weights, write_weights, usage_vec,
           precedence_weight, link_matrix, W_out, b_out):
    B, M, A = memory.shape
    R = read_weights.shape[2]
    OUT = W_out.shape[1]
    f32 = jnp.float32

    wkeyt = interface[:, 0:A].T
    wvec = interface[:, A:2 * A]
    erase = interface[:, 2 * A:3 * A]
    free = interface[:, 3 * A:3 * A + R]
    rstr = interface[:, 3 * A + R:3 * A + 2 * R]
    scal = interface[:, 3 * A + 2 * R:3 * A + 2 * R + 3]
    base = 3 * A + 2 * R + 3
    rkeys = interface[:, base:base + R * A].reshape(B, R, A).transpose(0, 2, 1)
    rmraw = interface[:, base + R * A:base + R * A + 3 * R] \
        .reshape(B, R, 3).transpose(0, 2, 1)

    memf = memory.reshape(B * M, A)
    rw0 = read_weights[:, :, 0]
    rw1 = read_weights[:, :, 1]
    rw2 = read_weights[:, :, 2]
    rw3 = read_weights[:, :, 3]
    ww2 = write_weights[:, :, 0]

    g1 = lambda arr: pl.BlockSpec(arr.shape, lambda: (0,) * arr.ndim)

    memnewf, write_w2, usage_out, prec_out, raddrt = pl.pallas_call(
        _state_kernel,
        grid=(),
        in_specs=[g1(wkeyt), g1(wvec), g1(erase), g1(free), g1(rstr),
                  g1(scal), g1(rkeys), g1(memf), g1(rw0), g1(rw1),
                  g1(rw2), g1(rw3), g1(ww2), g1(usage_vec),
                  g1(precedence_weight)],
        out_specs=[pl.BlockSpec((B * M, A), lambda: (0, 0)),
                   pl.BlockSpec((B, M), lambda: (0, 0)),
                   pl.BlockSpec((B, M), lambda: (0, 0)),
                   pl.BlockSpec((B, M), lambda: (0, 0)),
                   pl.BlockSpec((B, R, M), lambda: (0, 0, 0))],
        out_shape=[jax.ShapeDtypeStruct((B * M, A), f32),
                   jax.ShapeDtypeStruct((B, M), f32),
                   jax.ShapeDtypeStruct((B, M), f32),
                   jax.ShapeDtypeStruct((B, M), f32),
                   jax.ShapeDtypeStruct((B, R, M), f32)],
    )(wkeyt, wvec, erase, free, rstr, scal, rkeys, memf, rw0, rw1, rw2,
      rw3, ww2, usage_vec, precedence_weight)

    mem_new = memnewf.reshape(B, M, A)
    write_w = write_w2.reshape(B, M, 1)
    w_row = write_w2.reshape(B, 1, M)
    pw_row = precedence_weight.reshape(B, 1, M)
    W_perm = W_out.reshape(A, R, OUT).transpose(1, 0, 2)
    bout2 = b_out.reshape(1, OUT)
    nj = M // TJ

    full = lambda arr: pl.BlockSpec(arr.shape, lambda b, j: (0,) * arr.ndim)
    per_b = lambda *dims: pl.BlockSpec((1,) + dims,
                                       lambda b, j: (b,) + (0,) * len(dims))

    L_new, read_w, read_vec, mem_out = pl.pallas_call(
        _link_kernel,
        grid=(B, nj),
        in_specs=[pl.BlockSpec((1, TJ, M), lambda b, j: (b, j, 0)),
                  per_b(M, 1), per_b(1, M), per_b(1, M), per_b(M, R),
                  per_b(M, A), per_b(R, M), per_b(3, R),
                  full(W_perm), full(bout2)],
        out_specs=[pl.BlockSpec((1, TJ, M), lambda b, j: (b, j, 0)),
                   per_b(M, R), per_b(A, R), per_b(1, OUT)],
        out_shape=[jax.ShapeDtypeStruct((B, M, M), f32),
                   jax.ShapeDtypeStruct((B, M, R), f32),
                   jax.ShapeDtypeStruct((B, A, R), f32),
                   jax.ShapeDtypeStruct((B, 1, OUT), f32)],
        scratch_shapes=[pltpu.VMEM((M, R), f32),
                        pltpu.VMEM((R, M), f32)],
        compiler_params=pltpu.CompilerParams(
            dimension_semantics=("parallel", "arbitrary")),
    )(link_matrix, write_w, w_row, pw_row, read_weights, mem_new, raddrt,
      rmraw, W_perm, bout2)

    return (mem_out.reshape(B, OUT), mem_new, read_w, write_w, read_vec,
            usage_out, prec_out, L_new)


# speedup vs baseline: 1.2327x; 1.2327x over previous
import jax
import jax.numpy as jnp
from jax.experimental import pallas as pl
from jax.experimental.pallas import tpu as pltpu

EPS = 1e-6
TJ = 1024


def _state_kernel(wkeyt_ref, wvec_ref, erase_ref, free_ref, rstr_ref,
                  scal_ref, rkeys_ref, memf_ref, rw0_ref, rw1_ref, rw2_ref,
                  rw3_ref, ww_ref, usage_ref, pw_ref,
                  memnewf_ref, w2_ref, usageout_ref, precout_ref,
                  raddrt_ref):
    B, M = ww_ref.shape
    A = memf_ref.shape[1]
    R = rkeys_ref.shape[2]

    ws_col = jax.nn.softplus(scal_ref[:, 0:1]) + 1.0
    ag_col = jax.nn.sigmoid(scal_ref[:, 1:2])
    wg_col = jax.nn.sigmoid(scal_ref[:, 2:3])
    fg = jax.nn.sigmoid(free_ref[:])

    rws = (rw0_ref[:], rw1_ref[:], rw2_ref[:], rw3_ref[:])
    retention = 1.0 - fg[:, 0:1] * rws[0]
    for r in range(1, R):
        retention = retention * (1.0 - fg[:, r:r + 1] * rws[r])
    prev_u = usage_ref[:]
    ww = ww_ref[:]
    usage = ((prev_u + ww) - prev_u * ww) * retention
    usageout_ref[:] = usage

    u = EPS + (1.0 - EPS) * usage
    logu = jnp.log(u)
    ii = jax.lax.broadcasted_iota(jnp.int32, (M, M), 0)
    jj = jax.lax.broadcasted_iota(jnp.int32, (M, M), 1)
    ltm = jj < ii
    alloc_rows = []
    for b in range(B):
        u_row = u[b:b + 1, :]
        u_col = jnp.transpose(u_row)
        logu_col = jnp.transpose(logu[b:b + 1, :])
        mask = (u_row < u_col) | ((u_row == u_col) & ltm)
        s = jnp.dot(mask.astype(jnp.float32), logu_col,
                    preferred_element_type=jnp.float32)
        alloc_rows.append(jnp.transpose(s))
    s_all = jnp.concatenate(alloc_rows, axis=0)
    alloc = (1.0 - u) * jnp.exp(s_all)

    memf = memf_ref[:]
    dots_all = jnp.dot(memf, wkeyt_ref[:],
                       preferred_element_type=jnp.float32)
    key_norm = jnp.sqrt(jnp.sum(wkeyt_ref[:] * wkeyt_ref[:], axis=0,
                                keepdims=True))
    msq = jnp.sum(memf * memf, axis=1, keepdims=True)
    dot_rows, norm_rows = [], []
    for b in range(B):
        dot_rows.append(jnp.transpose(dots_all[b * M:(b + 1) * M, b:b + 1]))
        norm_rows.append(jnp.transpose(msq[b * M:(b + 1) * M, :]))
    dots = jnp.concatenate(dot_rows, axis=0)
    mem_norm = jnp.sqrt(jnp.concatenate(norm_rows, axis=0))
    sim = dots / (mem_norm * jnp.transpose(key_norm) + EPS) * ws_col
    write_addr = jax.nn.softmax(sim, axis=1)

    write_w = wg_col * ((1.0 - ag_col) * write_addr + ag_col * alloc)
    w2_ref[:] = write_w

    precout_ref[:] = ((1.0 - jnp.sum(write_w, axis=1, keepdims=True))
                      * pw_ref[:] + write_w)

    erase = jax.nn.sigmoid(erase_ref[:])
    rstr = jax.nn.softplus(rstr_ref[:]) + 1.0
    for b in range(B):
        w_col = jnp.transpose(write_w[b:b + 1, :])
        mem_b = memf[b * M:(b + 1) * M, :]
        mem_new = (mem_b * (1.0 - w_col * erase[b:b + 1, :])
                   + w_col * wvec_ref[b:b + 1, :])
        memnewf_ref[b * M:(b + 1) * M, :] = mem_new

        rkeys = rkeys_ref[b]
        dotr = jnp.dot(mem_new, rkeys,
                       preferred_element_type=jnp.float32)
        nsq = jnp.sum(mem_new * mem_new, axis=1, keepdims=True)
        dotr_t = jnp.transpose(dotr)
        norm_t = jnp.sqrt(jnp.transpose(nsq))
        rkn_col = jnp.sqrt(jnp.sum(rkeys * rkeys, axis=0,
                                   keepdims=True)).reshape(R, 1)
        simr_t = (dotr_t / (norm_t * rkn_col + EPS)
                  * jnp.transpose(rstr[b:b + 1, :]))
        raddrt_ref[b] = jax.nn.softmax(simr_t, axis=1)


def _link_kernel(link_ref, wcol_ref, wrow_ref, pwrow_ref, rw_ref,
                 memnew_ref, raddrt_ref, rmraw_ref, wperm_ref, bout_ref,
                 lnew_ref, readw_ref, readvec_ref, out_ref,
                 fwd_s, bwd_s):
    j = pl.program_id(1)
    nj = pl.num_programs(1)
    M = rw_ref.shape[1]
    R = rw_ref.shape[2]
    tj = link_ref.shape[1]

    L = link_ref[0]
    w_row = wrow_ref[0]
    pw_row = pwrow_ref[0]
    wJ = wcol_ref[0, pl.ds(j * tj, tj), :]
    rw_full = rw_ref[0]

    lnew = (1.0 - wJ + w_row) * L + wJ * pw_row
    row_g = jax.lax.broadcasted_iota(jnp.int32, (tj, M), 0) + j * tj
    col_g = jax.lax.broadcasted_iota(jnp.int32, (tj, M), 1)
    lnew = jnp.where(row_g == col_g, 0.0, lnew)
    lnew_ref[0] = lnew

    fwd_s[pl.ds(j * tj, tj), :] = jnp.dot(
        lnew, rw_full, preferred_element_type=jnp.float32)

    rwJ = rw_ref[0, pl.ds(j * tj, tj), :]
    contrib = jax.lax.dot_general(rwJ, lnew, (((0,), (0,)), ((), ())),
                                  preferred_element_type=jnp.float32)

    @pl.when(j == 0)
    def _():
        bwd_s[:] = contrib

    @pl.when(j != 0)
    def _():
        bwd_s[:] += contrib

    @pl.when(j == nj - 1)
    def _stage_c():
        rm = jax.nn.softmax(rmraw_ref[0], axis=0)
        rm_col = jnp.transpose(rm)
        read_w_t = (bwd_s[:] * rm_col[:, 0:1]
                    + raddrt_ref[0] * rm_col[:, 1:2]
                    + jnp.transpose(fwd_s[:]) * rm_col[:, 2:3])
        readw_ref[0] = jnp.transpose(read_w_t)
        rv_t = jax.lax.dot_general(read_w_t, memnew_ref[0],
                                   (((1,), (0,)), ((), ())),
                                   preferred_element_type=jnp.float32)
        readvec_ref[0] = jnp.transpose(rv_t)
        acc = bout_ref[:]
        for r in range(R):
            acc = acc + jax.lax.dot_general(
                rv_t[r:r + 1, :], wperm_ref[r],
                (((1,), (0,)), ((), ())),
                preferred_element_type=jnp.float32)
        out_ref[0] = acc


def kernel(interface, memory, read_weights, write_weights, usage_vec,
           precedence_weight, link_matrix, W_out, b_out):
    B, M, A = memory.shape
    R = read_weights.shape[2]
    OUT = W_out.shape[1]
    f32 = jnp.float32

    wkeyt = interface[:, 0:A].T
    wvec = interface[:, A:2 * A]
    erase = interface[:, 2 * A:3 * A]
    free = interface[:, 3 * A:3 * A + R]
    rstr = interface[:, 3 * A + R:3 * A + 2 * R]
    scal = interface[:, 3 * A + 2 * R:3 * A + 2 * R + 3]
    base = 3 * A + 2 * R + 3
    rkeys = interface[:, base:base + R * A].reshape(B, R, A).transpose(0, 2, 1)
    rmraw = interface[:, base + R * A:base + R * A + 3 * R] \
        .reshape(B, R, 3).transpose(0, 2, 1)

    memf = memory.reshape(B * M, A)
    rw0 = read_weights[:, :, 0]
    rw1 = read_weights[:, :, 1]
    rw2 = read_weights[:, :, 2]
    rw3 = read_weights[:, :, 3]
    ww2 = write_weights[:, :, 0]

    g1 = lambda arr: pl.BlockSpec(arr.shape, lambda: (0,) * arr.ndim)

    memnewf, write_w2, usage_out, prec_out, raddrt = pl.pallas_call(
        _state_kernel,
        grid=(),
        in_specs=[g1(wkeyt), g1(wvec), g1(erase), g1(free), g1(rstr),
                  g1(scal), g1(rkeys), g1(memf), g1(rw0), g1(rw1),
                  g1(rw2), g1(rw3), g1(ww2), g1(usage_vec),
                  g1(precedence_weight)],
        out_specs=[pl.BlockSpec((B * M, A), lambda: (0, 0)),
                   pl.BlockSpec((B, M), lambda: (0, 0)),
                   pl.BlockSpec((B, M), lambda: (0, 0)),
                   pl.BlockSpec((B, M), lambda: (0, 0)),
                   pl.BlockSpec((B, R, M), lambda: (0, 0, 0))],
        out_shape=[jax.ShapeDtypeStruct((B * M, A), f32),
                   jax.ShapeDtypeStruct((B, M), f32),
                   jax.ShapeDtypeStruct((B, M), f32),
                   jax.ShapeDtypeStruct((B, M), f32),
                   jax.ShapeDtypeStruct((B, R, M), f32)],
    )(wkeyt, wvec, erase, free, rstr, scal, rkeys, memf, rw0, rw1, rw2,
      rw3, ww2, usage_vec, precedence_weight)

    mem_new = memnewf.reshape(B, M, A)
    write_w = write_w2.reshape(B, M, 1)
    w_row = write_w2.reshape(B, 1, M)
    pw_row = precedence_weight.reshape(B, 1, M)
    W_perm = W_out.reshape(A, R, OUT).transpose(1, 0, 2)
    bout2 = b_out.reshape(1, OUT)
    nj = M // TJ

    full = lambda arr: pl.BlockSpec(arr.shape, lambda b, j: (0,) * arr.ndim)
    per_b = lambda *dims: pl.BlockSpec((1,) + dims,
                                       lambda b, j: (b,) + (0,) * len(dims))

    L_new, read_w, read_vec, mem_out = pl.pallas_call(
        _link_kernel,
        grid=(B, nj),
        in_specs=[pl.BlockSpec((1, TJ, M), lambda b, j: (b, j, 0)),
                  per_b(M, 1), per_b(1, M), per_b(1, M), per_b(M, R),
                  per_b(M, A), per_b(R, M), per_b(3, R),
                  full(W_perm), full(bout2)],
        out_specs=[pl.BlockSpec((1, TJ, M), lambda b, j: (b, j, 0)),
                   per_b(M, R), per_b(A, R), per_b(1, OUT)],
        out_shape=[jax.ShapeDtypeStruct((B, M, M), f32),
                   jax.ShapeDtypeStruct((B, M, R), f32),
                   jax.ShapeDtypeStruct((B, A, R), f32),
                   jax.ShapeDtypeStruct((B, 1, OUT), f32)],
        scratch_shapes=[pltpu.VMEM((M, R), f32),
                        pltpu.VMEM((R, M), f32)],
        compiler_params=pltpu.CompilerParams(
            dimension_semantics=("parallel", "arbitrary")),
    )(link_matrix, write_w, w_row, pw_row, read_weights, mem_new, raddrt,
      rmraw, W_perm, bout2)

    return (mem_out.reshape(B, OUT), mem_new, read_w, write_w, read_vec,
            usage_out, prec_out, L_new)


# speedup vs baseline: 1.2434x; 1.0087x over previous
import jax
import jax.numpy as jnp
from jax.experimental import pallas as pl
from jax.experimental.pallas import tpu as pltpu

EPS = 1e-6
TJ = 1024


def _state_kernel(wkeyt_ref, wvec_ref, erase_ref, free_ref, rstr_ref,
                  scal_ref, rkeys_ref, memf_ref, rw0_ref, rw1_ref, rw2_ref,
                  rw3_ref, ww_ref, usage_ref, pw_ref,
                  memnewf_ref, w2_ref, usageout_ref, precout_ref,
                  raddrt_ref):
    B, M = ww_ref.shape
    A = memf_ref.shape[1]
    R = rkeys_ref.shape[2]

    ws_col = jax.nn.softplus(scal_ref[:, 0:1]) + 1.0
    ag_col = jax.nn.sigmoid(scal_ref[:, 1:2])
    wg_col = jax.nn.sigmoid(scal_ref[:, 2:3])
    fg = jax.nn.sigmoid(free_ref[:])

    rws = (rw0_ref[:], rw1_ref[:], rw2_ref[:], rw3_ref[:])
    retention = 1.0 - fg[:, 0:1] * rws[0]
    for r in range(1, R):
        retention = retention * (1.0 - fg[:, r:r + 1] * rws[r])
    prev_u = usage_ref[:]
    ww = ww_ref[:]
    usage = ((prev_u + ww) - prev_u * ww) * retention
    usageout_ref[:] = usage

    u = EPS + (1.0 - EPS) * usage
    logu = jnp.log(u)
    ii = jax.lax.broadcasted_iota(jnp.int32, (M, M), 0)
    jj = jax.lax.broadcasted_iota(jnp.int32, (M, M), 1)
    ltm = jj < ii
    alloc_rows = []
    for b in range(B):
        u_row = u[b:b + 1, :]
        u_col = jnp.transpose(u_row)
        logu_col = jnp.transpose(logu[b:b + 1, :])
        mask = (u_row < u_col) | ((u_row == u_col) & ltm)
        s = jnp.dot(mask.astype(jnp.float32), logu_col,
                    preferred_element_type=jnp.float32)
        alloc_rows.append(jnp.transpose(s))
    s_all = jnp.concatenate(alloc_rows, axis=0)
    alloc = (1.0 - u) * jnp.exp(s_all)

    memf = memf_ref[:]
    dots_all = jnp.dot(memf, wkeyt_ref[:],
                       preferred_element_type=jnp.float32)
    key_norm = jnp.sqrt(jnp.sum(wkeyt_ref[:] * wkeyt_ref[:], axis=0,
                                keepdims=True))
    msq = jnp.sum(memf * memf, axis=1, keepdims=True)
    dot_rows, norm_rows = [], []
    for b in range(B):
        dot_rows.append(jnp.transpose(dots_all[b * M:(b + 1) * M, b:b + 1]))
        norm_rows.append(jnp.transpose(msq[b * M:(b + 1) * M, :]))
    dots = jnp.concatenate(dot_rows, axis=0)
    mem_norm = jnp.sqrt(jnp.concatenate(norm_rows, axis=0))
    sim = dots / (mem_norm * jnp.transpose(key_norm) + EPS) * ws_col
    write_addr = jax.nn.softmax(sim, axis=1)

    write_w = wg_col * ((1.0 - ag_col) * write_addr + ag_col * alloc)
    w2_ref[:] = write_w

    precout_ref[:] = ((1.0 - jnp.sum(write_w, axis=1, keepdims=True))
                      * pw_ref[:] + write_w)

    erase = jax.nn.sigmoid(erase_ref[:])
    rstr = jax.nn.softplus(rstr_ref[:]) + 1.0
    for b in range(B):
        w_col = jnp.transpose(write_w[b:b + 1, :])
        mem_b = memf[b * M:(b + 1) * M, :]
        mem_new = (mem_b * (1.0 - w_col * erase[b:b + 1, :])
                   + w_col * wvec_ref[b:b + 1, :])
        memnewf_ref[b * M:(b + 1) * M, :] = mem_new

        rkeys = rkeys_ref[b]
        dotr = jnp.dot(mem_new, rkeys,
                       preferred_element_type=jnp.float32)
        nsq = jnp.sum(mem_new * mem_new, axis=1, keepdims=True)
        dotr_t = jnp.transpose(dotr)
        norm_t = jnp.sqrt(jnp.transpose(nsq))
        rkn_col = jnp.sqrt(jnp.sum(rkeys * rkeys, axis=0,
                                   keepdims=True)).reshape(R, 1)
        simr_t = (dotr_t / (norm_t * rkn_col + EPS)
                  * jnp.transpose(rstr[b:b + 1, :]))
        raddrt_ref[b] = jax.nn.softmax(simr_t, axis=1)


def _link_kernel(link_ref, wcol_ref, wrow_ref, pwrow_ref, rw_ref,
                 memnew_ref, raddrt_ref, rmraw_ref, wperm_ref, bout_ref,
                 lnew_ref, readw_ref, readvec_ref, out_ref):
    M = rw_ref.shape[1]
    R = rw_ref.shape[2]

    L = link_ref[0]
    w_row = wrow_ref[0]
    pw_row = pwrow_ref[0]
    w_col = wcol_ref[0]
    rw_full = rw_ref[0]

    lnew = (1.0 - w_col + w_row) * L + w_col * pw_row
    row_g = jax.lax.broadcasted_iota(jnp.int32, (M, M), 0)
    col_g = jax.lax.broadcasted_iota(jnp.int32, (M, M), 1)
    lnew = jnp.where(row_g == col_g, 0.0, lnew)
    lnew_ref[0] = lnew

    fwd = jnp.dot(lnew, rw_full,
                  preferred_element_type=jnp.float32)
    bwd_t = jax.lax.dot_general(rw_full, lnew, (((0,), (0,)), ((), ())),
                                preferred_element_type=jnp.float32)

    rm = jax.nn.softmax(rmraw_ref[0], axis=0)
    rm_col = jnp.transpose(rm)
    read_w_t = (bwd_t * rm_col[:, 0:1]
                + raddrt_ref[0] * rm_col[:, 1:2]
                + jnp.transpose(fwd) * rm_col[:, 2:3])
    readw_ref[0] = jnp.transpose(read_w_t)
    rv_t = jax.lax.dot_general(read_w_t, memnew_ref[0],
                               (((1,), (0,)), ((), ())),
                               preferred_element_type=jnp.float32)
    readvec_ref[0] = jnp.transpose(rv_t)
    acc = bout_ref[:]
    for r in range(R):
        acc = acc + jax.lax.dot_general(
            rv_t[r:r + 1, :], wperm_ref[r],
            (((1,), (0,)), ((), ())),
            preferred_element_type=jnp.float32)
    out_ref[0] = acc


def kernel(interface, memory, read_weights, write_weights, usage_vec,
           precedence_weight, link_matrix, W_out, b_out):
    B, M, A = memory.shape
    R = read_weights.shape[2]
    OUT = W_out.shape[1]
    f32 = jnp.float32

    wkeyt = interface[:, 0:A].T
    wvec = interface[:, A:2 * A]
    erase = interface[:, 2 * A:3 * A]
    free = interface[:, 3 * A:3 * A + R]
    rstr = interface[:, 3 * A + R:3 * A + 2 * R]
    scal = interface[:, 3 * A + 2 * R:3 * A + 2 * R + 3]
    base = 3 * A + 2 * R + 3
    rkeys = interface[:, base:base + R * A].reshape(B, R, A).transpose(0, 2, 1)
    rmraw = interface[:, base + R * A:base + R * A + 3 * R] \
        .reshape(B, R, 3).transpose(0, 2, 1)

    memf = memory.reshape(B * M, A)
    rw0 = read_weights[:, :, 0]
    rw1 = read_weights[:, :, 1]
    rw2 = read_weights[:, :, 2]
    rw3 = read_weights[:, :, 3]
    ww2 = write_weights[:, :, 0]

    g1 = lambda arr: pl.BlockSpec(arr.shape, lambda: (0,) * arr.ndim)

    memnewf, write_w2, usage_out, prec_out, raddrt = pl.pallas_call(
        _state_kernel,
        grid=(),
        in_specs=[g1(wkeyt), g1(wvec), g1(erase), g1(free), g1(rstr),
                  g1(scal), g1(rkeys), g1(memf), g1(rw0), g1(rw1),
                  g1(rw2), g1(rw3), g1(ww2), g1(usage_vec),
                  g1(precedence_weight)],
        out_specs=[pl.BlockSpec((B * M, A), lambda: (0, 0)),
                   pl.BlockSpec((B, M), lambda: (0, 0)),
                   pl.BlockSpec((B, M), lambda: (0, 0)),
                   pl.BlockSpec((B, M), lambda: (0, 0)),
                   pl.BlockSpec((B, R, M), lambda: (0, 0, 0))],
        out_shape=[jax.ShapeDtypeStruct((B * M, A), f32),
                   jax.ShapeDtypeStruct((B, M), f32),
                   jax.ShapeDtypeStruct((B, M), f32),
                   jax.ShapeDtypeStruct((B, M), f32),
                   jax.ShapeDtypeStruct((B, R, M), f32)],
    )(wkeyt, wvec, erase, free, rstr, scal, rkeys, memf, rw0, rw1, rw2,
      rw3, ww2, usage_vec, precedence_weight)

    mem_new = memnewf.reshape(B, M, A)
    write_w = write_w2.reshape(B, M, 1)
    w_row = write_w2.reshape(B, 1, M)
    pw_row = precedence_weight.reshape(B, 1, M)
    W_perm = W_out.reshape(A, R, OUT).transpose(1, 0, 2)
    bout2 = b_out.reshape(1, OUT)

    full = lambda arr: pl.BlockSpec(arr.shape, lambda b: (0,) * arr.ndim)
    per_b = lambda *dims: pl.BlockSpec((1,) + dims,
                                       lambda b: (b,) + (0,) * len(dims))

    L_new, read_w, read_vec, mem_out = pl.pallas_call(
        _link_kernel,
        grid=(B,),
        in_specs=[per_b(M, M),
                  per_b(M, 1), per_b(1, M), per_b(1, M), per_b(M, R),
                  per_b(M, A), per_b(R, M), per_b(3, R),
                  full(W_perm), full(bout2)],
        out_specs=[per_b(M, M),
                   per_b(M, R), per_b(A, R), per_b(1, OUT)],
        out_shape=[jax.ShapeDtypeStruct((B, M, M), f32),
                   jax.ShapeDtypeStruct((B, M, R), f32),
                   jax.ShapeDtypeStruct((B, A, R), f32),
                   jax.ShapeDtypeStruct((B, 1, OUT), f32)],
        compiler_params=pltpu.CompilerParams(
            dimension_semantics=("parallel",)),
    )(link_matrix, write_w, w_row, pw_row, read_weights, mem_new, raddrt,
      rmraw, W_perm, bout2)

    return (mem_out.reshape(B, OUT), mem_new, read_w, write_w, read_vec,
            usage_out, prec_out, L_new)


# speedup vs baseline: 1.2742x; 1.0247x over previous
import jax
import jax.numpy as jnp
from jax.experimental import pallas as pl
from jax.experimental.pallas import tpu as pltpu

EPS = 1e-6
TJ = 1024


def _state_kernel(wkeyt_ref, wvec_ref, erase_ref, free_ref, rstr_ref,
                  scal_ref, rkeys_ref, memf_ref, rw0_ref, rw1_ref, rw2_ref,
                  rw3_ref, ww_ref, usage_ref, pw_ref,
                  memnewf_ref, w2_ref, usageout_ref, precout_ref,
                  raddrt_ref):
    B, M = ww_ref.shape
    A = memf_ref.shape[1]
    R = rkeys_ref.shape[2]

    ws_col = jax.nn.softplus(scal_ref[:, 0:1]) + 1.0
    ag_col = jax.nn.sigmoid(scal_ref[:, 1:2])
    wg_col = jax.nn.sigmoid(scal_ref[:, 2:3])
    fg = jax.nn.sigmoid(free_ref[:])

    rws = (rw0_ref[:], rw1_ref[:], rw2_ref[:], rw3_ref[:])
    retention = 1.0 - fg[:, 0:1] * rws[0]
    for r in range(1, R):
        retention = retention * (1.0 - fg[:, r:r + 1] * rws[r])
    prev_u = usage_ref[:]
    ww = ww_ref[:]
    usage = ((prev_u + ww) - prev_u * ww) * retention
    usageout_ref[:] = usage

    u = EPS + (1.0 - EPS) * usage
    logu = jnp.log(u)
    u_cols = jnp.transpose(u)
    logu_cols = jnp.transpose(logu)
    ii = jax.lax.broadcasted_iota(jnp.int32, (M, M), 0)
    jj = jax.lax.broadcasted_iota(jnp.int32, (M, M), 1)
    ltm = jj < ii
    s_cols = []
    for b in range(B):
        u_row = u[b:b + 1, :]
        u_col = u_cols[:, b:b + 1]
        mask = (u_row < u_col) | ((u_row == u_col) & ltm)
        s_cols.append(jnp.dot(mask.astype(jnp.float32),
                              logu_cols[:, b:b + 1],
                              preferred_element_type=jnp.float32))
    s_all = jnp.transpose(jnp.concatenate(s_cols, axis=1))
    alloc = (1.0 - u) * jnp.exp(s_all)

    memf = memf_ref[:]
    dots_all = jnp.dot(memf, wkeyt_ref[:],
                       preferred_element_type=jnp.float32)
    key_norm = jnp.sqrt(jnp.sum(wkeyt_ref[:] * wkeyt_ref[:], axis=0,
                                keepdims=True))
    msq = jnp.sum(memf * memf, axis=1, keepdims=True)
    dot_rows, norm_rows = [], []
    for b in range(B):
        dot_rows.append(jnp.transpose(dots_all[b * M:(b + 1) * M, b:b + 1]))
        norm_rows.append(jnp.transpose(msq[b * M:(b + 1) * M, :]))
    dots = jnp.concatenate(dot_rows, axis=0)
    mem_norm = jnp.sqrt(jnp.concatenate(norm_rows, axis=0))
    sim = dots / (mem_norm * jnp.transpose(key_norm) + EPS) * ws_col
    write_addr = jax.nn.softmax(sim, axis=1)

    write_w = wg_col * ((1.0 - ag_col) * write_addr + ag_col * alloc)
    w2_ref[:] = write_w

    precout_ref[:] = ((1.0 - jnp.sum(write_w, axis=1, keepdims=True))
                      * pw_ref[:] + write_w)

    erase = jax.nn.sigmoid(erase_ref[:])
    rstr = jax.nn.softplus(rstr_ref[:]) + 1.0
    for b in range(B):
        w_col = jnp.transpose(write_w[b:b + 1, :])
        mem_b = memf[b * M:(b + 1) * M, :]
        mem_new = (mem_b * (1.0 - w_col * erase[b:b + 1, :])
                   + w_col * wvec_ref[b:b + 1, :])
        memnewf_ref[b * M:(b + 1) * M, :] = mem_new

        rkeys = rkeys_ref[b]
        dotr = jnp.dot(mem_new, rkeys,
                       preferred_element_type=jnp.float32)
        nsq = jnp.sum(mem_new * mem_new, axis=1, keepdims=True)
        dotr_t = jnp.transpose(dotr)
        norm_t = jnp.sqrt(jnp.transpose(nsq))
        rkn_col = jnp.sqrt(jnp.sum(rkeys * rkeys, axis=0,
                                   keepdims=True)).reshape(R, 1)
        simr_t = (dotr_t / (norm_t * rkn_col + EPS)
                  * jnp.transpose(rstr[b:b + 1, :]))
        raddrt_ref[b] = jax.nn.softmax(simr_t, axis=1)


def _link_kernel(link_ref, wcol_ref, wrow_ref, pwrow_ref, rw_ref,
                 memnew_ref, raddrt_ref, rmraw_ref, wperm_ref, bout_ref,
                 lnew_ref, readw_ref, readvec_ref, out_ref):
    M = rw_ref.shape[1]
    R = rw_ref.shape[2]

    L = link_ref[0]
    w_row = wrow_ref[0]
    pw_row = pwrow_ref[0]
    w_col = wcol_ref[0]
    rw_full = rw_ref[0]

    lnew = (1.0 - w_col + w_row) * L + w_col * pw_row
    row_g = jax.lax.broadcasted_iota(jnp.int32, (M, M), 0)
    col_g = jax.lax.broadcasted_iota(jnp.int32, (M, M), 1)
    lnew = jnp.where(row_g == col_g, 0.0, lnew)
    lnew_ref[0] = lnew

    fwd = jnp.dot(lnew, rw_full,
                  preferred_element_type=jnp.float32)
    bwd_t = jax.lax.dot_general(rw_full, lnew, (((0,), (0,)), ((), ())),
                                preferred_element_type=jnp.float32)

    rm = jax.nn.softmax(rmraw_ref[0], axis=0)
    rm_col = jnp.transpose(rm)
    read_w_t = (bwd_t * rm_col[:, 0:1]
                + raddrt_ref[0] * rm_col[:, 1:2]
                + jnp.transpose(fwd) * rm_col[:, 2:3])
    readw_ref[0] = jnp.transpose(read_w_t)
    rv_t = jax.lax.dot_general(read_w_t, memnew_ref[0],
                               (((1,), (0,)), ((), ())),
                               preferred_element_type=jnp.float32)
    readvec_ref[0] = jnp.transpose(rv_t)
    acc = bout_ref[:]
    for r in range(R):
        acc = acc + jax.lax.dot_general(
            rv_t[r:r + 1, :], wperm_ref[r],
            (((1,), (0,)), ((), ())),
            preferred_element_type=jnp.float32)
    out_ref[0] = acc


def kernel(interface, memory, read_weights, write_weights, usage_vec,
           precedence_weight, link_matrix, W_out, b_out):
    B, M, A = memory.shape
    R = read_weights.shape[2]
    OUT = W_out.shape[1]
    f32 = jnp.float32

    wkeyt = interface[:, 0:A].T
    wvec = interface[:, A:2 * A]
    erase = interface[:, 2 * A:3 * A]
    free = interface[:, 3 * A:3 * A + R]
    rstr = interface[:, 3 * A + R:3 * A + 2 * R]
    scal = interface[:, 3 * A + 2 * R:3 * A + 2 * R + 3]
    base = 3 * A + 2 * R + 3
    rkeys = interface[:, base:base + R * A].reshape(B, R, A).transpose(0, 2, 1)
    rmraw = interface[:, base + R * A:base + R * A + 3 * R] \
        .reshape(B, R, 3).transpose(0, 2, 1)

    memf = memory.reshape(B * M, A)
    rw0 = read_weights[:, :, 0]
    rw1 = read_weights[:, :, 1]
    rw2 = read_weights[:, :, 2]
    rw3 = read_weights[:, :, 3]
    ww2 = write_weights[:, :, 0]

    g1 = lambda arr: pl.BlockSpec(arr.shape, lambda: (0,) * arr.ndim)

    memnewf, write_w2, usage_out, prec_out, raddrt = pl.pallas_call(
        _state_kernel,
        grid=(),
        in_specs=[g1(wkeyt), g1(wvec), g1(erase), g1(free), g1(rstr),
                  g1(scal), g1(rkeys), g1(memf), g1(rw0), g1(rw1),
                  g1(rw2), g1(rw3), g1(ww2), g1(usage_vec),
                  g1(precedence_weight)],
        out_specs=[pl.BlockSpec((B * M, A), lambda: (0, 0)),
                   pl.BlockSpec((B, M), lambda: (0, 0)),
                   pl.BlockSpec((B, M), lambda: (0, 0)),
                   pl.BlockSpec((B, M), lambda: (0, 0)),
                   pl.BlockSpec((B, R, M), lambda: (0, 0, 0))],
        out_shape=[jax.ShapeDtypeStruct((B * M, A), f32),
                   jax.ShapeDtypeStruct((B, M), f32),
                   jax.ShapeDtypeStruct((B, M), f32),
                   jax.ShapeDtypeStruct((B, M), f32),
                   jax.ShapeDtypeStruct((B, R, M), f32)],
    )(wkeyt, wvec, erase, free, rstr, scal, rkeys, memf, rw0, rw1, rw2,
      rw3, ww2, usage_vec, precedence_weight)

    mem_new = memnewf.reshape(B, M, A)
    write_w = write_w2.reshape(B, M, 1)
    w_row = write_w2.reshape(B, 1, M)
    pw_row = precedence_weight.reshape(B, 1, M)
    W_perm = W_out.reshape(A, R, OUT).transpose(1, 0, 2)
    bout2 = b_out.reshape(1, OUT)

    full = lambda arr: pl.BlockSpec(arr.shape, lambda b: (0,) * arr.ndim)
    per_b = lambda *dims: pl.BlockSpec((1,) + dims,
                                       lambda b: (b,) + (0,) * len(dims))

    L_new, read_w, read_vec, mem_out = pl.pallas_call(
        _link_kernel,
        grid=(B,),
        in_specs=[per_b(M, M),
                  per_b(M, 1), per_b(1, M), per_b(1, M), per_b(M, R),
                  per_b(M, A), per_b(R, M), per_b(3, R),
                  full(W_perm), full(bout2)],
        out_specs=[per_b(M, M),
                   per_b(M, R), per_b(A, R), per_b(1, OUT)],
        out_shape=[jax.ShapeDtypeStruct((B, M, M), f32),
                   jax.ShapeDtypeStruct((B, M, R), f32),
                   jax.ShapeDtypeStruct((B, A, R), f32),
                   jax.ShapeDtypeStruct((B, 1, OUT), f32)],
        compiler_params=pltpu.CompilerParams(
            dimension_semantics=("parallel",)),
    )(link_matrix, write_w, w_row, pw_row, read_weights, mem_new, raddrt,
      rmraw, W_perm, bout2)

    return (mem_out.reshape(B, OUT), mem_new, read_w, write_w, read_vec,
            usage_out, prec_out, L_new)
